# Initial kernel scaffold; baseline (speedup 1.0000x reference)
import jax
import jax.numpy as jnp
from jax.experimental import pallas as pl


def kernel(boxes, image_meta, feature_map_p2, feature_map_p3, feature_map_p4, feature_map_p5):
    raise NotImplementedError("write your pallas kernel here")


# speedup vs baseline: 8.3193x; 8.3193x over previous
import functools

import jax
import jax.numpy as jnp
from jax import lax
from jax.experimental import pallas as pl
from jax.experimental.pallas import tpu as pltpu
from jax.experimental.pallas import tpu_sc as plsc

NC, NS, L = 2, 16, 16
NW = NC * NS
N_BOX = 1000
BOX_PER_W = 32
PH, PW = 7, 7
PIX = PH * PW
ROWS = PIX * 4
HALF = 104
RPAD = 2 * HALF
C = 256
CV = C // L
OUT_ROWS = N_BOX * PIX
IDX_PER_W = BOX_PER_W * RPAD
WTS_PER_W = BOX_PER_W * PIX * L
OPIX = PIX * C


def _routing(boxes, image_meta, sizes):
    n = boxes.shape[0] * boxes.shape[1]
    y1, x1, y2, x2 = jnp.split(boxes, 4, axis=2)
    h = y2 - y1
    w = x2 - x1
    image_shape = image_meta[0, 4:7]
    image_area = (image_shape[0] * image_shape[1]).astype(jnp.float32)
    roi_level = jnp.log(jnp.sqrt(h * w) / (224.0 / jnp.sqrt(image_area))) / jnp.log(2.0)
    roi_level = jnp.minimum(5, jnp.maximum(2, 4 + jnp.round(roi_level).astype(jnp.int32)))
    li = jnp.squeeze(roi_level, 2).reshape(-1) - 2

    sz = jnp.array(sizes, jnp.int32)
    bases = jnp.array([0] + list(jnp.cumsum(jnp.array([s * s for s in sizes]))[:-1]), jnp.int32)
    side = sz[li]
    base = bases[li]
    sm1f = (side - 1).astype(jnp.float32)[:, None]
    sm1i = (side - 1)[:, None]

    fb = boxes.reshape(n, 4)
    by1, bx1, by2, bx2 = fb[:, 0:1], fb[:, 1:2], fb[:, 2:3], fb[:, 3:4]
    t = (jnp.arange(PH, dtype=jnp.float32) / (PH - 1))[None, :]
    ys = by1 * sm1f + t * ((by2 - by1) * sm1f)
    xs = bx1 * sm1f + t * ((bx2 - bx1) * sm1f)
    y0f = jnp.floor(ys)
    x0f = jnp.floor(xs)
    ly = ys - y0f
    lx = xs - x0f
    y0 = jnp.clip(y0f.astype(jnp.int32), 0, sm1i)
    yi1 = jnp.clip(y0 + 1, 0, sm1i)
    x0 = jnp.clip(x0f.astype(jnp.int32), 0, sm1i)
    xi1 = jnp.clip(x0 + 1, 0, sm1i)

    wb = side[:, None, None]
    bb = base[:, None, None]
    yy0 = y0[:, :, None] * wb
    yy1 = yi1[:, :, None] * wb
    xx0 = x0[:, None, :]
    xx1 = xi1[:, None, :]
    idx = jnp.stack(
        [bb + yy0 + xx0, bb + yy0 + xx1, bb + yy1 + xx0, bb + yy1 + xx1],
        axis=-1,
    ).reshape(n, ROWS).astype(jnp.int32)

    wy0 = (1.0 - ly)[:, :, None]
    wy1 = ly[:, :, None]
    wx0 = (1.0 - lx)[:, None, :]
    wx1 = lx[:, None, :]
    valid = (((ys >= 0) & (ys <= sm1f))[:, :, None]
             & ((xs >= 0) & (xs <= sm1f))[:, None, :]).astype(jnp.float32)[..., None]
    wts = (jnp.stack(
        [wy0 * wx0, wy0 * wx1, wy1 * wx0, wy1 * wx1], axis=-1
    ) * valid).reshape(n, ROWS)
    return idx, wts


def _sc_gather_kernel():
    mesh = plsc.VectorSubcoreMesh(
        core_axis_name="c", subcore_axis_name="s", num_cores=NC, num_subcores=NS
    )

    @functools.partial(
        pl.kernel,
        out_type=jax.ShapeDtypeStruct((OUT_ROWS * C,), jnp.float32),
        mesh=mesh,
        scratch_types=[
            pltpu.VMEM((IDX_PER_W,), jnp.int32),
            pltpu.VMEM((WTS_PER_W,), jnp.float32),
            pltpu.VMEM((RPAD, C), jnp.float32),
            pltpu.VMEM((OPIX,), jnp.float32),
            pltpu.SemaphoreType.DMA,
        ],
    )
    def k(table, idx_hbm, w_hbm, out, idx_v, w_v, rows_v, obuf, sem):
        wid = lax.axis_index("s") * NC + lax.axis_index("c")
        pltpu.sync_copy(idx_hbm.at[pl.ds(wid * IDX_PER_W, IDX_PER_W)], idx_v)
        pltpu.sync_copy(w_hbm.at[pl.ds(wid * WTS_PER_W, WTS_PER_W)], w_v)

        def box_body(b, carry):
            gbox = wid * BOX_PER_W + b

            @pl.when(gbox < N_BOX)
            def _():
                cp0 = pltpu.async_copy(
                    table.at[idx_v.at[pl.ds(b * RPAD, HALF)]],
                    rows_v.at[pl.ds(0, HALF)], sem,
                )
                cp1 = pltpu.async_copy(
                    table.at[idx_v.at[pl.ds(b * RPAD + HALF, HALF)]],
                    rows_v.at[pl.ds(HALF, HALF)], sem,
                )
                cp0.wait()
                cp1.wait()

                def pix_body(p, carry2):
                    r = 4 * p
                    wv = w_v[pl.ds((b * PIX + p) * L, L)]
                    w0 = jnp.full((L,), wv[0], jnp.float32)
                    w1 = jnp.full((L,), wv[1], jnp.float32)
                    w2 = jnp.full((L,), wv[2], jnp.float32)
                    w3 = jnp.full((L,), wv[3], jnp.float32)
                    for cb in range(CV):
                        s = pl.ds(cb * L, L)
                        acc = rows_v[r, s] * w0
                        acc = acc + rows_v[r + 1, s] * w1
                        acc = acc + rows_v[r + 2, s] * w2
                        acc = acc + rows_v[r + 3, s] * w3
                        obuf[pl.ds(p * C + cb * L, L)] = acc
                    return carry2

                lax.fori_loop(0, PIX, pix_body, 0, unroll=False)
                pltpu.sync_copy(obuf, out.at[pl.ds(gbox * OPIX, OPIX)])

            return carry

        lax.fori_loop(0, BOX_PER_W, box_body, 0, unroll=False)

    return k


def kernel(boxes, image_meta, feature_map_p2, feature_map_p3, feature_map_p4, feature_map_p5):
    fmaps = [feature_map_p2, feature_map_p3, feature_map_p4, feature_map_p5]
    sizes = [m.shape[1] for m in fmaps]
    b, n = boxes.shape[0], boxes.shape[1]

    idx, wts = _routing(boxes, image_meta, sizes)
    pad = NW * BOX_PER_W - b * n
    idx = jnp.pad(idx, ((0, pad), (0, RPAD - ROWS))).reshape(-1)
    wts = jnp.pad(wts.reshape(-1, PIX, 4), ((0, pad), (0, 0), (0, L - 4))).reshape(-1)
    table = jnp.concatenate([m.reshape(-1, C) for m in fmaps], axis=0)

    out = _sc_gather_kernel()(table, idx, wts)
    return out.reshape(b, n, PH, PW, C)


# speedup vs baseline: 13.4322x; 1.6146x over previous
import functools

import jax
import jax.numpy as jnp
from jax import lax
from jax.experimental import pallas as pl
from jax.experimental.pallas import tpu as pltpu
from jax.experimental.pallas import tpu_sc as plsc

NC, NS, L = 2, 16, 16
NW = NC * NS
N_BOX = 1000
SLOTS = 32
NBIG = N_BOX - 31 * NW
PH, PW = 7, 7
PIX = PH * PW
ROWS = PIX * 4
HALF = 104
PIX_A, PIX_B = 26, 23
RPAD = 2 * HALF
LVL_SLOT = 196
C = 256
CV = C // L
OUT_ROWS = N_BOX * PIX
WSTRIDE = 8
IDX_PER_W = SLOTS * RPAD
WTS_PER_W = SLOTS * PIX * WSTRIDE + 8
OPIX = PIX * C


def _routing(boxes, image_meta, sizes):
    n = boxes.shape[0] * boxes.shape[1]
    y1, x1, y2, x2 = jnp.split(boxes, 4, axis=2)
    h = y2 - y1
    w = x2 - x1
    image_shape = image_meta[0, 4:7]
    image_area = (image_shape[0] * image_shape[1]).astype(jnp.float32)
    roi_level = jnp.log(jnp.sqrt(h * w) / (224.0 / jnp.sqrt(image_area))) / jnp.log(2.0)
    roi_level = jnp.minimum(5, jnp.maximum(2, 4 + jnp.round(roi_level).astype(jnp.int32)))
    li = jnp.squeeze(roi_level, 2).reshape(-1) - 2

    sz = jnp.array(sizes, jnp.int32)
    side = sz[li]
    sm1f = (side - 1).astype(jnp.float32)[:, None]
    sm1i = (side - 1)[:, None]

    fb = boxes.reshape(n, 4)
    by1, bx1, by2, bx2 = fb[:, 0:1], fb[:, 1:2], fb[:, 2:3], fb[:, 3:4]
    t = (jnp.arange(PH, dtype=jnp.float32) / (PH - 1))[None, :]
    ys = by1 * sm1f + t * ((by2 - by1) * sm1f)
    xs = bx1 * sm1f + t * ((bx2 - bx1) * sm1f)
    y0f = jnp.floor(ys)
    x0f = jnp.floor(xs)
    ly = ys - y0f
    lx = xs - x0f
    y0 = jnp.clip(y0f.astype(jnp.int32), 0, sm1i)
    yi1 = jnp.clip(y0 + 1, 0, sm1i)
    x0 = jnp.clip(x0f.astype(jnp.int32), 0, sm1i)
    xi1 = jnp.clip(x0 + 1, 0, sm1i)

    wb = side[:, None, None]
    yy0 = y0[:, :, None] * wb
    yy1 = yi1[:, :, None] * wb
    xx0 = x0[:, None, :]
    xx1 = xi1[:, None, :]
    idx = jnp.stack(
        [yy0 + xx0, yy0 + xx1, yy1 + xx0, yy1 + xx1], axis=-1
    ).reshape(n, ROWS).astype(jnp.int32)

    wy0 = (1.0 - ly)[:, :, None]
    wy1 = ly[:, :, None]
    wx0 = (1.0 - lx)[:, None, :]
    wx1 = lx[:, None, :]
    valid = (((ys >= 0) & (ys <= sm1f))[:, :, None]
             & ((xs >= 0) & (xs <= sm1f))[:, None, :]).astype(jnp.float32)[..., None]
    wts = (jnp.stack(
        [wy0 * wx0, wy0 * wx1, wy1 * wx0, wy1 * wx1], axis=-1
    ) * valid).reshape(n, PIX, 4)
    return idx, wts, li


def _sc_gather_kernel():
    mesh = plsc.VectorSubcoreMesh(
        core_axis_name="c", subcore_axis_name="s", num_cores=NC, num_subcores=NS
    )

    @functools.partial(
        pl.kernel,
        out_type=jax.ShapeDtypeStruct((OUT_ROWS * C,), jnp.float32),
        mesh=mesh,
        scratch_types=[
            pltpu.VMEM((IDX_PER_W,), jnp.int32),
            pltpu.VMEM((WTS_PER_W,), jnp.float32),
            pltpu.VMEM((HALF, C), jnp.float32),
            pltpu.VMEM((HALF, C), jnp.float32),
            pltpu.VMEM((OPIX,), jnp.float32),
            pltpu.SemaphoreType.DMA,
            pltpu.SemaphoreType.DMA,
        ],
    )
    def k(t2, t3, t4, t5, idx_hbm, w_hbm, out,
          idx_v, w_v, buf_a, buf_b, obuf, sem_a, sem_b):
        tables = (t2, t3, t4, t5)
        wid = lax.axis_index("s") * NC + lax.axis_index("c")
        cnt = jnp.where(wid < NBIG, SLOTS, SLOTS - 1)
        start = jnp.where(wid < NBIG, SLOTS * wid, (SLOTS - 1) * wid + NBIG)
        pltpu.sync_copy(idx_hbm.at[pl.ds(wid * IDX_PER_W, IDX_PER_W)], idx_v)
        pltpu.sync_copy(w_hbm.at[pl.ds(wid * WTS_PER_W, WTS_PER_W)], w_v)

        def issue(b, half, buf, sem):
            lvl = idx_v[pl.ds(b * RPAD + LVL_SLOT - 4, L)][4]
            idx_sl = idx_v.at[pl.ds(b * RPAD + half * HALF, HALF)]
            for l, tab in enumerate(tables):
                @pl.when(lvl == l)
                def _(tab=tab):
                    pltpu.async_copy(tab.at[idx_sl], buf, sem)

        def wait(buf, sem):
            pltpu.make_async_copy(t2.at[idx_v.at[pl.ds(0, HALF)]], buf, sem).wait()

        def blend(b, p0, npix, buf):
            def pix(i, carry):
                p = p0 + i
                r = 4 * i
                wv = w_v[pl.ds((b * PIX + p) * WSTRIDE, L)]
                w0 = jnp.full((L,), wv[0], jnp.float32)
                w1 = jnp.full((L,), wv[1], jnp.float32)
                w2 = jnp.full((L,), wv[2], jnp.float32)
                w3 = jnp.full((L,), wv[3], jnp.float32)
                for cb in range(CV):
                    s = pl.ds(cb * L, L)
                    acc = buf[r, s] * w0
                    acc = acc + buf[r + 1, s] * w1
                    acc = acc + buf[r + 2, s] * w2
                    acc = acc + buf[r + 3, s] * w3
                    obuf[pl.ds(p * C + cb * L, L)] = acc
                return carry

            lax.fori_loop(0, npix, pix, 0, unroll=False)

        issue(0, 0, buf_a, sem_a)
        issue(0, 1, buf_b, sem_b)

        def box_body(b, carry):
            wait(buf_a, sem_a)
            blend(b, 0, PIX_A, buf_a)

            @pl.when(b + 1 < cnt)
            def _():
                issue(b + 1, 0, buf_a, sem_a)

            wait(buf_b, sem_b)
            blend(b, PIX_A, PIX_B, buf_b)

            @pl.when(b + 1 < cnt)
            def _():
                issue(b + 1, 1, buf_b, sem_b)

            pltpu.sync_copy(obuf, out.at[pl.ds((start + b) * OPIX, OPIX)])
            return carry

        lax.fori_loop(0, cnt, box_body, 0, unroll=False)

    return k


def kernel(boxes, image_meta, feature_map_p2, feature_map_p3, feature_map_p4, feature_map_p5):
    fmaps = [feature_map_p2, feature_map_p3, feature_map_p4, feature_map_p5]
    sizes = [m.shape[1] for m in fmaps]
    b, n = boxes.shape[0], boxes.shape[1]

    idx, wts, li = _routing(boxes, image_meta, sizes)
    idx = jnp.pad(idx, ((0, 0), (0, RPAD - ROWS)))
    idx = idx.at[:, LVL_SLOT].set(li)
    wts = jnp.pad(wts, ((0, 0), (0, 0), (0, WSTRIDE - 4))).reshape(-1, PIX * WSTRIDE)

    t = jnp.arange(NW, dtype=jnp.int32)
    starts = jnp.where(t < NBIG, SLOTS * t, (SLOTS - 1) * t + NBIG)
    slot_map = jnp.clip(starts[:, None] + jnp.arange(SLOTS, dtype=jnp.int32)[None, :],
                        0, b * n - 1).reshape(-1)
    idx = idx[slot_map].reshape(NW, SLOTS * RPAD).reshape(-1)
    wts = jnp.pad(wts[slot_map].reshape(NW, SLOTS * PIX * WSTRIDE),
                  ((0, 0), (0, 8))).reshape(-1)
    tables = [m.reshape(-1, C) for m in fmaps]

    out = _sc_gather_kernel()(*tables, idx, wts)
    return out.reshape(b, n, PH, PW, C)
